# Initial kernel scaffold; baseline (speedup 1.0000x reference)
import jax
import jax.numpy as jnp
from jax.experimental import pallas as pl


def kernel(x, matrix, params, batch, edge_index):
    raise NotImplementedError("write your pallas kernel here")


# speedup vs baseline: 2.7910x; 2.7910x over previous
import functools

import jax
import jax.numpy as jnp
from jax import lax
from jax.experimental import pallas as pl
from jax.experimental.pallas import tpu as pltpu
from jax.experimental.pallas import tpu_sc as plsc

N = 10000
E = 160000
B = 128

NC = 2
NS = 16
NW = NC * NS
PER_W = E // NW
ROWS_PER_SUB = N // NS

TILE_E = 2000
TILE_N = 2000


def _lsilu(x, alpha):
    return x * (1.0 / (1.0 + jnp.exp(-x)) + alpha)


def _bn_fold(bn):
    k = bn["g"] / jnp.sqrt(bn["rv"] + 1e-5)
    return k, bn["be"] - bn["rm"] * k


def _fold_conv(p, din):
    k1, s1 = _bn_fold(p["bn1"])
    w1 = p["l1"]["w"] * k1[None, :]
    b1 = p["l1"]["b"] * k1 + s1
    k2, s2 = _bn_fold(p["bn2"])
    w2 = p["l2"]["w"] * k2[None, :]
    b2 = p["l2"]["b"] * k2 + s2
    return w1[:din], w1[din:], b1[None, :], w2, b2[None, :]


def _embed_call(x, batch_col, m0, m1, m2):
    grid = N // TILE_N

    def body(x_ref, b_ref, m0_ref, m1_ref, m2_ref, o_ref):
        xt = x_ref[...]
        bt = b_ref[...]
        iota = lax.broadcasted_iota(jnp.int32, (1, B), 1).astype(jnp.float32)
        oh = jnp.where(bt == iota, 1.0, 0.0)
        col0 = lax.broadcasted_iota(jnp.int32, (1, 4), 1) == 0
        acc = xt[:, 0:1] * jnp.where(col0, 1.0, 0.0)
        for k, m_ref in enumerate((m0_ref, m1_ref, m2_ref)):
            mk = jnp.dot(oh, m_ref[...], preferred_element_type=jnp.float32)
            acc = acc + xt[:, 1 + k:2 + k] * mk
        o_ref[...] = acc

    return pl.pallas_call(
        body,
        grid=(grid,),
        in_specs=[
            pl.BlockSpec((TILE_N, 4), lambda i: (i, 0)),
            pl.BlockSpec((TILE_N, 1), lambda i: (i, 0)),
            pl.BlockSpec((B, 4), lambda i: (0, 0)),
            pl.BlockSpec((B, 4), lambda i: (0, 0)),
            pl.BlockSpec((B, 4), lambda i: (0, 0)),
        ],
        out_specs=pl.BlockSpec((TILE_N, 4), lambda i: (i, 0)),
        out_shape=jax.ShapeDtypeStruct((N, 4), jnp.float32),
    )(x, batch_col, m0, m1, m2)


def _edge_mlp_call(xd, xs, w1d, w1s, b1, w2, b2):
    din = xd.shape[1]
    h_dim = w1d.shape[1]
    dout = w2.shape[1]
    grid = E // TILE_E

    def body(xd_ref, xs_ref, w1d_ref, w1s_ref, b1_ref, w2_ref, b2_ref, o_ref):
        h = jnp.dot(xd_ref[...], w1d_ref[...], preferred_element_type=jnp.float32)
        h = h + jnp.dot(xs_ref[...], w1s_ref[...], preferred_element_type=jnp.float32)
        h = _lsilu(h + b1_ref[...], 0.05)
        m = jnp.dot(h, w2_ref[...], preferred_element_type=jnp.float32) + b2_ref[...]
        o_ref[...] = _lsilu(m, 0.05)

    return pl.pallas_call(
        body,
        grid=(grid,),
        in_specs=[
            pl.BlockSpec((TILE_E, din), lambda i: (i, 0)),
            pl.BlockSpec((TILE_E, din), lambda i: (i, 0)),
            pl.BlockSpec((din, h_dim), lambda i: (0, 0)),
            pl.BlockSpec((din, h_dim), lambda i: (0, 0)),
            pl.BlockSpec((1, h_dim), lambda i: (0, 0)),
            pl.BlockSpec((h_dim, dout), lambda i: (0, 0)),
            pl.BlockSpec((1, dout), lambda i: (0, 0)),
        ],
        out_specs=pl.BlockSpec((TILE_E, dout), lambda i: (i, 0)),
        out_shape=jax.ShapeDtypeStruct((E, dout), jnp.float32),
    )(xd, xs, w1d, w1s, b1, w2, b2)


def _node_update_call(part, cnt, w1, b1, w2, b2, scale, shift):
    d = part.shape[2]
    h_dim = w1.shape[1]
    grid = N // TILE_N

    def body(p_ref, c_ref, w1_ref, b1_ref, w2_ref, b2_ref, sc_ref, sh_ref, o_ref):
        s = p_ref[0] + p_ref[1]
        c = c_ref[0, :, 0:1] + c_ref[1, :, 0:1]
        mean = s * (1.0 / jnp.maximum(c, 1.0))
        nf0 = _lsilu(mean, 0.1)
        h = _lsilu(jnp.dot(nf0, w1_ref[...], preferred_element_type=jnp.float32)
                   + b1_ref[...], 0.05)
        t = _lsilu(jnp.dot(h, w2_ref[...], preferred_element_type=jnp.float32)
                   + b2_ref[...], 0.05)
        o_ref[...] = t * sc_ref[...] + sh_ref[...] + nf0

    return pl.pallas_call(
        body,
        grid=(grid,),
        in_specs=[
            pl.BlockSpec((2, TILE_N, d), lambda i: (0, i, 0)),
            pl.BlockSpec((2, TILE_N, 8), lambda i: (0, i, 0)),
            pl.BlockSpec((d, h_dim), lambda i: (0, 0)),
            pl.BlockSpec((1, h_dim), lambda i: (0, 0)),
            pl.BlockSpec((h_dim, d), lambda i: (0, 0)),
            pl.BlockSpec((1, d), lambda i: (0, 0)),
            pl.BlockSpec((1, d), lambda i: (0, 0)),
            pl.BlockSpec((1, d), lambda i: (0, 0)),
        ],
        out_specs=pl.BlockSpec((TILE_N, d), lambda i: (i, 0)),
        out_shape=jax.ShapeDtypeStruct((N, d), jnp.float32),
    )(part, cnt, w1, b1, w2, b2, scale, shift)


def _pool_fc_call(part, cnt, batch_row, fw1, fb1, fw2, fb2):
    d = part.shape[2]
    grid = N // TILE_N

    def body(p_ref, c_ref, b_ref, fw1_ref, fb1_ref, fw2_ref, fb2_ref, o_ref, acc):
        i = pl.program_id(0)
        s = p_ref[0] + p_ref[1]
        c = c_ref[0, :, 0:1] + c_ref[1, :, 0:1]
        nf = _lsilu(s * (1.0 / jnp.maximum(c, 1.0)), 0.1)
        bt = b_ref[0]
        iota = lax.broadcasted_iota(jnp.int32, (B, 1), 0).astype(jnp.float32)
        oh_t = jnp.where(iota == bt, 1.0, 0.0)
        g = jnp.dot(oh_t, nf, preferred_element_type=jnp.float32)

        @pl.when(i == 0)
        def _():
            acc[...] = jnp.zeros_like(acc)

        acc[...] += g

        @pl.when(i == grid - 1)
        def _():
            h = jnp.dot(acc[...], fw1_ref[...], preferred_element_type=jnp.float32)
            h = _lsilu(h + fb1_ref[...], 0.1)
            o_ref[...] = (jnp.dot(h, fw2_ref[...], preferred_element_type=jnp.float32)
                          + fb2_ref[...] - 100.0)

    return pl.pallas_call(
        body,
        grid=(grid,),
        in_specs=[
            pl.BlockSpec((2, TILE_N, d), lambda i: (0, i, 0)),
            pl.BlockSpec((2, TILE_N, 8), lambda i: (0, i, 0)),
            pl.BlockSpec((1, 1, TILE_N), lambda i: (i, 0, 0)),
            pl.BlockSpec((d, d), lambda i: (0, 0)),
            pl.BlockSpec((1, d), lambda i: (0, 0)),
            pl.BlockSpec((d, 1), lambda i: (0, 0)),
            pl.BlockSpec((1, 1), lambda i: (0, 0)),
        ],
        out_specs=pl.BlockSpec((B, 1), lambda i: (0, 0)),
        out_shape=jax.ShapeDtypeStruct((B, 1), jnp.float32),
        scratch_shapes=[pltpu.VMEM((B, d), jnp.float32)],
    )(part, cnt, batch_row, fw1, fb1, fw2, fb2)


def _sc_gather(nf, src, dst):
    d = nf.shape[1]
    chunk = 1000 if d <= 32 else 200
    iters = PER_W // chunk
    mesh = plsc.VectorSubcoreMesh(core_axis_name="c", subcore_axis_name="s")

    @functools.partial(
        pl.kernel,
        out_type=(jax.ShapeDtypeStruct((E, d), jnp.float32),
                  jax.ShapeDtypeStruct((E, d), jnp.float32)),
        mesh=mesh,
        compiler_params=pltpu.CompilerParams(use_tc_tiling_on_sc=False),
        scratch_types=[
            pltpu.VMEM((chunk,), jnp.int32),
            pltpu.VMEM((chunk,), jnp.int32),
            pltpu.VMEM((chunk, d), jnp.float32),
            pltpu.VMEM((chunk, d), jnp.float32),
            pltpu.SemaphoreType.DMA,
            pltpu.SemaphoreType.DMA,
        ],
    )
    def gk(nf_hbm, src_hbm, dst_hbm, xd_out, xs_out,
           idx_d, idx_s, rows_d, rows_s, sem_d, sem_s):
        wid = lax.axis_index("s") * NC + lax.axis_index("c")
        base = wid * PER_W

        def body(i, carry):
            off = base + i * chunk
            pltpu.sync_copy(dst_hbm.at[pl.ds(off, chunk)], idx_d)
            pltpu.sync_copy(src_hbm.at[pl.ds(off, chunk)], idx_s)
            cp_d = pltpu.async_copy(nf_hbm.at[idx_d], rows_d, sem_d)
            cp_s = pltpu.async_copy(nf_hbm.at[idx_s], rows_s, sem_s)
            cp_d.wait()
            cp_s.wait()
            pltpu.sync_copy(rows_d, xd_out.at[pl.ds(off, chunk)])
            pltpu.sync_copy(rows_s, xs_out.at[pl.ds(off, chunk)])
            return carry

        lax.fori_loop(0, iters, body, 0)

    return gk(nf, src, dst)


def _sc_scatter(msg, dst, zeros):
    d = msg.shape[1]
    chunk = 1000 if d <= 32 else 200
    iters = PER_W // chunk
    mesh = plsc.VectorSubcoreMesh(core_axis_name="c", subcore_axis_name="s")

    @functools.partial(
        pl.kernel,
        out_type=jax.ShapeDtypeStruct((NC, N, d), jnp.float32),
        mesh=mesh,
        compiler_params=pltpu.CompilerParams(use_tc_tiling_on_sc=False),
        scratch_types=[
            pltpu.VMEM((chunk,), jnp.int32),
            pltpu.VMEM((chunk, d), jnp.float32),
            pltpu.VMEM_SHARED((N, d), jnp.float32),
        ],
    )
    def sk(msg_hbm, dst_hbm, z_hbm, out_hbm, idx_v, rows_v, acc):
        cid = lax.axis_index("c")
        sid = lax.axis_index("s")
        wid = sid * NC + cid
        r0 = sid * ROWS_PER_SUB
        pltpu.sync_copy(z_hbm.at[pl.ds(r0, ROWS_PER_SUB)],
                        acc.at[pl.ds(r0, ROWS_PER_SUB)])
        plsc.subcore_barrier()
        base = wid * PER_W

        def body(i, carry):
            off = base + i * chunk
            pltpu.sync_copy(dst_hbm.at[pl.ds(off, chunk)], idx_v)
            pltpu.sync_copy(msg_hbm.at[pl.ds(off, chunk)], rows_v)
            pltpu.sync_copy(rows_v, acc.at[idx_v], add=True)
            return carry

        lax.fori_loop(0, iters, body, 0)
        plsc.subcore_barrier()
        pltpu.sync_copy(acc.at[pl.ds(r0, ROWS_PER_SUB)],
                        out_hbm.at[cid, pl.ds(r0, ROWS_PER_SUB)])

    return sk(msg, dst, zeros)


def _sc_counts(dst, ones, zeros):
    chunk = 1000
    iters = PER_W // chunk
    mesh = plsc.VectorSubcoreMesh(core_axis_name="c", subcore_axis_name="s")

    @functools.partial(
        pl.kernel,
        out_type=jax.ShapeDtypeStruct((NC, N, 8), jnp.float32),
        mesh=mesh,
        compiler_params=pltpu.CompilerParams(use_tc_tiling_on_sc=False),
        scratch_types=[
            pltpu.VMEM((chunk,), jnp.int32),
            pltpu.VMEM((chunk, 8), jnp.float32),
            pltpu.VMEM_SHARED((N, 8), jnp.float32),
        ],
    )
    def ck(dst_hbm, ones_hbm, z_hbm, out_hbm, idx_v, ones_v, acc):
        cid = lax.axis_index("c")
        sid = lax.axis_index("s")
        wid = sid * NC + cid
        r0 = sid * ROWS_PER_SUB
        pltpu.sync_copy(ones_hbm, ones_v)
        pltpu.sync_copy(z_hbm.at[pl.ds(r0, ROWS_PER_SUB)],
                        acc.at[pl.ds(r0, ROWS_PER_SUB)])
        plsc.subcore_barrier()
        base = wid * PER_W

        def body(i, carry):
            off = base + i * chunk
            pltpu.sync_copy(dst_hbm.at[pl.ds(off, chunk)], idx_v)
            pltpu.sync_copy(ones_v, acc.at[idx_v], add=True)
            return carry

        lax.fori_loop(0, iters, body, 0)
        plsc.subcore_barrier()
        pltpu.sync_copy(acc.at[pl.ds(r0, ROWS_PER_SUB)],
                        out_hbm.at[cid, pl.ds(r0, ROWS_PER_SUB)])

    return ck(dst, ones, zeros)


def _conv_layer(nf, src, dst, cnt, p, din):
    w1d, w1s, b1, w2, b2 = _fold_conv(p, din)
    xd, xs = _sc_gather(nf, src, dst)
    msg = _edge_mlp_call(xd, xs, w1d, w1s, b1, w2, b2)
    dout = msg.shape[1]
    part = _sc_scatter(msg, dst, jnp.zeros((N, dout), jnp.float32))
    return part


def kernel(x, matrix, params, batch, edge_index):
    src = edge_index[0]
    dst = edge_index[1]
    batch_f = batch.astype(jnp.float32)
    batch_col = batch_f.reshape(N, 1)
    batch_row = batch_f.reshape(N // TILE_N, 1, TILE_N)

    mg = matrix.astype(jnp.float32).reshape(B, 3, 3)
    zpad = jnp.zeros((B, 1), jnp.float32)
    m0 = jnp.concatenate([zpad, mg[:, 0, :]], axis=1)
    m1 = jnp.concatenate([zpad, mg[:, 1, :]], axis=1)
    m2 = jnp.concatenate([zpad, mg[:, 2, :]], axis=1)

    nf = _embed_call(x, batch_col, m0, m1, m2)

    cnt = _sc_counts(dst, jnp.ones((1000, 8), jnp.float32),
                     jnp.zeros((N, 8), jnp.float32))

    dims = (4, 32, 128, 64, 128)
    for li in range(4):
        p = params[f"conv{li + 1}"]
        part = _conv_layer(nf, src, dst, cnt, p, dims[li])
        ip = params[f"il{li + 1}"]
        k, s = _bn_fold(ip["bn"])
        nf = _node_update_call(
            part, cnt,
            ip["l1"]["w"], ip["l1"]["b"][None, :],
            ip["l2"]["w"], ip["l2"]["b"][None, :],
            k[None, :], s[None, :])

    part = _conv_layer(nf, src, dst, cnt, params["conv5"], dims[4])
    return _pool_fc_call(part, cnt, batch_row,
                         params["fc1"]["w"], params["fc1"]["b"][None, :],
                         params["fc2"]["w"], params["fc2"]["b"][None, :])
